# baseline (speedup 1.0000x reference)
import functools

import jax
import jax.numpy as jnp
from jax import lax
from jax.experimental import pallas as pl
from jax.experimental.pallas import tpu as pltpu
from jax.experimental.pallas import tpu_sc as plsc

RANK = 32
BATCH = 16384

_INFO = plsc.get_sparse_core_info()
_NC = _INFO.num_cores
_NS = _INFO.num_subcores
_NW = _NC * _NS
_BPW = BATCH // _NW
_CHUNK = 128
_NCHUNK = _BPW // _CHUNK


def _sc_gather_body(tidx_hbm, ridx_hbm, cidx_hbm, tt_hbm, ut_hbm, it_hbm,
                    out_t, out_u, out_i,
                    ti_v, ri_v, ci_v, tr_v, ur_v, ir_v, sem):
    wid = lax.axis_index("s") * _NC + lax.axis_index("c")
    base = wid * _BPW
    for j in range(_NCHUNK):
        off = pl.ds(base + j * _CHUNK, _CHUNK)
        pltpu.sync_copy(tidx_hbm.at[off], ti_v.at[j])
        pltpu.sync_copy(ridx_hbm.at[off], ri_v.at[j])
        pltpu.sync_copy(cidx_hbm.at[off], ci_v.at[j])
    copies = []
    for j in range(_NCHUNK):
        dst = pl.ds(j * _CHUNK, _CHUNK)
        copies.append(pltpu.async_copy(tt_hbm.at[ti_v.at[j]], tr_v.at[dst], sem))
        copies.append(pltpu.async_copy(ut_hbm.at[ri_v.at[j]], ur_v.at[dst], sem))
        copies.append(pltpu.async_copy(it_hbm.at[ci_v.at[j]], ir_v.at[dst], sem))
    for c in copies:
        c.wait()
    out_slice = pl.ds(base, _BPW)
    pltpu.sync_copy(tr_v, out_t.at[out_slice])
    pltpu.sync_copy(ur_v, out_u.at[out_slice])
    pltpu.sync_copy(ir_v, out_i.at[out_slice])


@jax.jit
def _sc_gather(tIdx, rIdx, cIdx, time_table, user_table, item_table):
    mesh = plsc.VectorSubcoreMesh(core_axis_name="c", subcore_axis_name="s")
    f = functools.partial(
        pl.kernel,
        mesh=mesh,
        compiler_params=pltpu.CompilerParams(use_tc_tiling_on_sc=False),
        out_type=(
            jax.ShapeDtypeStruct((BATCH, RANK), jnp.float32),
            jax.ShapeDtypeStruct((BATCH, RANK), jnp.float32),
            jax.ShapeDtypeStruct((BATCH, RANK), jnp.float32),
        ),
        scratch_types=[
            pltpu.VMEM((_NCHUNK, _CHUNK), jnp.int32),
            pltpu.VMEM((_NCHUNK, _CHUNK), jnp.int32),
            pltpu.VMEM((_NCHUNK, _CHUNK), jnp.int32),
            pltpu.VMEM((_BPW, RANK), jnp.float32),
            pltpu.VMEM((_BPW, RANK), jnp.float32),
            pltpu.VMEM((_BPW, RANK), jnp.float32),
            pltpu.SemaphoreType.DMA,
        ],
    )(_sc_gather_body)
    return f(tIdx, rIdx, cIdx, time_table, user_table, item_table)


def _tc_body(et_ref, eu_ref, ei_ref, wt_ref, wu_ref, wi_ref,
             bt_ref, bu_ref, bi_ref, o_ref):
    t = jnp.dot(et_ref[...], wt_ref[...], preferred_element_type=jnp.float32) + bt_ref[...]
    u = jnp.dot(eu_ref[...], wu_ref[...], preferred_element_type=jnp.float32) + bu_ref[...]
    i = jnp.dot(ei_ref[...], wi_ref[...], preferred_element_type=jnp.float32) + bi_ref[...]
    o_ref[...] = jnp.sum(t * u * i, axis=-1)


_TC_BLOCK = 2048


@jax.jit
def _tc_combine(et, eu, ei, WtT, WuT, WiT, bt, bu, bi):
    grid = BATCH // _TC_BLOCK
    emb_spec = pl.BlockSpec((_TC_BLOCK, RANK), lambda i: (i, 0))
    w_spec = pl.BlockSpec((RANK, RANK), lambda i: (0, 0))
    b_spec = pl.BlockSpec((1, RANK), lambda i: (0, 0))
    return pl.pallas_call(
        _tc_body,
        grid=(grid,),
        in_specs=[emb_spec, emb_spec, emb_spec, w_spec, w_spec, w_spec,
                  b_spec, b_spec, b_spec],
        out_specs=pl.BlockSpec((_TC_BLOCK,), lambda i: (i,)),
        out_shape=jax.ShapeDtypeStruct((BATCH,), jnp.float32),
    )(et, eu, ei, WtT, WuT, WiT, bt, bu, bi)


def kernel(tIdx, rIdx, cIdx, time_table, user_table, item_table,
           Wt, bt, Wu, bu, Wi, bi):
    et, eu, ei = _sc_gather(tIdx, rIdx, cIdx, time_table, user_table, item_table)
    return _tc_combine(et, eu, ei, Wt.T, Wu.T, Wi.T,
                       bt.reshape(1, RANK), bu.reshape(1, RANK),
                       bi.reshape(1, RANK))
